# baseline (speedup 1.0000x reference)
import jax
import jax.numpy as jnp
from jax.experimental import pallas as pl
from jax.experimental.pallas import tpu as pltpu

_S_BLK = 2048
_E_BLK = 512


def _body(x_ref, pe_ref, o_ref):
    o_ref[...] = x_ref[...] + pe_ref[...]


def kernel(x, pe):
    B, S, E = x.shape
    grid = (S // _S_BLK, E // _E_BLK, B)
    return pl.pallas_call(
        _body,
        grid=grid,
        in_specs=[
            pl.BlockSpec((1, _S_BLK, _E_BLK), lambda i, e, b: (b, i, e)),
            pl.BlockSpec((_S_BLK, _E_BLK), lambda i, e, b: (i, e)),
        ],
        out_specs=pl.BlockSpec((1, _S_BLK, _E_BLK), lambda i, e, b: (b, i, e)),
        out_shape=jax.ShapeDtypeStruct(x.shape, x.dtype),
    )(x, pe)
